# baseline (speedup 1.0000x reference)
import functools

import jax
import jax.numpy as jnp
from jax import lax
from jax.experimental import pallas as pl
from jax.experimental.pallas import tpu as pltpu
from jax.experimental.pallas import tpu_sc as plsc

NNODES = 10000
NP = 10112
EMB = 32
NW = 32
EPW = 10000
CHUNK = 128
NCH = 80
EPWP = NCH * CHUNK
B = 1024
SEQ = 50
SPW = 1600
SCHUNK = 80
SNCH = SPW // SCHUNK
DGR = EPW // 16

_SC_MESH = plsc.VectorSubcoreMesh(
    core_axis_name="c", subcore_axis_name="s", num_cores=2, num_subcores=16)
_SC_PARAMS = pltpu.CompilerParams(
    needs_layout_passes=False, use_tc_tiling_on_sc=False)


@functools.partial(
    pl.kernel,
    out_type=(jax.ShapeDtypeStruct((NW, NP), jnp.float32),
              jax.ShapeDtypeStruct((B * SEQ, EMB), jnp.float32)),
    mesh=_SC_MESH,
    compiler_params=_SC_PARAMS,
    scratch_types=[pltpu.VMEM((DGR, 16), jnp.int32),
                   pltpu.VMEM((NP,), jnp.float32),
                   pltpu.VMEM((SPW,), jnp.int32),
                   pltpu.VMEM((SPW, EMB), jnp.float32),
                   pltpu.SemaphoreType.DMA])
def _sc_degemb(dstr_hbm, seqt_hbm, table_hbm, deg_hbm, emb_hbm,
               dst_v, deg_v, idx_v, rows_v, sem):
    c = lax.axis_index("c")
    s = lax.axis_index("s")
    wid = c * 16 + s

    pltpu.sync_copy(seqt_hbm.at[wid], idx_v)
    descs = [
        pltpu.async_copy(table_hbm.at[idx_v.at[pl.ds(k * SCHUNK, SCHUNK)]],
                         rows_v.at[pl.ds(k * SCHUNK, SCHUNK)], sem)
        for k in range(SNCH)
    ]

    def zero(i, carry):
        deg_v[pl.ds(i * 16, 16)] = jnp.zeros((16,), jnp.float32)
        return carry
    lax.fori_loop(0, NP // 16, zero, 0)

    pltpu.sync_copy(dstr_hbm.at[wid], dst_v)
    ones = jnp.ones((16,), jnp.float32)

    def body(g, carry):
        plsc.addupdate_scatter(deg_v, [dst_v[g]], ones)
        return carry
    lax.fori_loop(0, DGR, body, 0)
    pltpu.sync_copy(deg_v, deg_hbm.at[wid])

    for d in descs:
        d.wait()
    pltpu.sync_copy(rows_v, emb_hbm.at[pl.ds(wid * SPW, SPW)])


@functools.partial(
    pl.kernel,
    out_type=(jax.ShapeDtypeStruct((2, NP, EMB), jnp.float32),
              jax.ShapeDtypeStruct((NW, NP), jnp.float32)),
    mesh=_SC_MESH,
    compiler_params=_SC_PARAMS,
    scratch_types=[
        pltpu.VMEM((EPWP,), jnp.int32),
        pltpu.VMEM((NCH, CHUNK), jnp.int32),
        pltpu.VMEM((NP,), jnp.float32),
        pltpu.VMEM((NP,), jnp.float32),
        pltpu.VMEM((CHUNK, EMB), jnp.float32),
        pltpu.VMEM((CHUNK, EMB), jnp.float32),
        pltpu.VMEM((CHUNK, EMB), jnp.float32),
        pltpu.VMEM((CHUNK, EMB), jnp.float32),
        pltpu.VMEM((NP // 16, EMB), jnp.float32),
        pltpu.VMEM_SHARED((NP, EMB), jnp.float32),
        pltpu.SemaphoreType.DMA, pltpu.SemaphoreType.DMA,
        pltpu.SemaphoreType.DMA, pltpu.SemaphoreType.DMA,
        pltpu.SemaphoreType.DMA, pltpu.SemaphoreType.DMA,
        pltpu.SemaphoreType.DMA, pltpu.SemaphoreType.DMA,
    ])
def _sc_edges(srcf_hbm, dstt_hbm, y_hbm, dinv_hbm,
              seg_hbm, cpart_hbm,
              src_v, dstt_v, dinv_v, c_v, r0, r1, r2, r3, dr_v, seg_sh,
              g0, g1, g2, g3, s0, s1, s2, s3):
    c = lax.axis_index("c")
    s = lax.axis_index("s")
    wid = c * 16 + s
    npt = NP // 16
    bufs = (r0, r1, r2, r3)
    gsems = (g0, g1, g2, g3)
    ssems = (s0, s1, s2, s3)

    def zero_c(i, carry):
        c_v[pl.ds(i * 16, 16)] = jnp.zeros((16,), jnp.float32)
        return carry
    lax.fori_loop(0, NP // 16, zero_c, 0)

    def zero_dr(i, carry):
        dr_v[i // 2, pl.ds((i % 2) * 16, 16)] = jnp.zeros((16,), jnp.float32)
        return carry
    lax.fori_loop(0, npt * 2, zero_dr, 0)
    pltpu.sync_copy(dr_v, seg_sh.at[pl.ds(s * npt, npt)])

    pltpu.sync_copy(srcf_hbm.at[wid], src_v)
    pltpu.sync_copy(dstt_hbm.at[wid], dstt_v)
    pltpu.sync_copy(dinv_hbm, dinv_v)
    plsc.subcore_barrier()

    for b in range(2):
        pltpu.async_copy(
            y_hbm.at[src_v.at[pl.ds(b * CHUNK, CHUNK)]], bufs[b], gsems[b])

    def quad(k, carry):
        for b in range(4):
            j = k * 4 + b
            bn = (b + 2) % 4
            pltpu.make_async_copy(
                y_hbm.at[src_v.at[pl.ds(j * CHUNK, CHUNK)]],
                bufs[b], gsems[b]).wait()
            pltpu.make_async_copy(
                bufs[b], seg_sh.at[dstt_v.at[j]], ssems[b]).start(add=True)
            for g in range(CHUNK // 16):
                base = j * CHUNK + g * 16
                di = plsc.load_gather(dinv_v, [dstt_v[j, pl.ds(g * 16, 16)]])
                plsc.addupdate_scatter(c_v, [src_v[pl.ds(base, 16)]], di)

            @pl.when(j >= 2)
            def _():
                pltpu.make_async_copy(
                    bufs[bn], seg_sh.at[dstt_v.at[j - 2]], ssems[bn]).wait()

            @pl.when(j + 2 < NCH)
            def _():
                pltpu.async_copy(
                    y_hbm.at[src_v.at[pl.ds((j + 2) * CHUNK, CHUNK)]],
                    bufs[bn], gsems[bn])
        return carry
    lax.fori_loop(0, NCH // 4, quad, 0)

    for b in range(2, 4):
        pltpu.make_async_copy(
            bufs[b], seg_sh.at[dstt_v.at[NCH - 4 + b]], ssems[b]).wait()

    plsc.subcore_barrier()
    pltpu.sync_copy(seg_sh.at[pl.ds(s * npt, npt)], dr_v)
    pltpu.sync_copy(dr_v, seg_hbm.at[c, pl.ds(s * npt, npt)])
    pltpu.sync_copy(c_v, cpart_hbm.at[wid])


def _t2_body(x_ref, w_ref, deg_ref, tab_ref, wt_ref, bt_ref,
             src_ref, dst_ref,
             y_ref, dinv_ref, tabemb_ref, srcp_ref, dstp_ref):
    ones = jnp.ones((NW, 1), jnp.float32)
    deg_col = lax.dot_general(
        deg_ref[...], ones, (((0,), (0,)), ((), ()))) + 1.0
    dinv_col = lax.rsqrt(deg_col)
    dinv_ref[...] = dinv_col
    xw = x_ref[...] @ w_ref[...]
    y_ref[0:NNODES, :] = xw * dinv_col[0:NNODES]
    y_ref[NNODES:NP, :] = jnp.zeros((NP - NNODES, EMB), jnp.float32)
    tabemb_ref[...] = jnp.maximum(tab_ref[...] @ wt_ref[...] + bt_ref[...], 0.0)
    fill = jnp.full((NW, EPWP - EPW), NNODES, jnp.int32)
    srcp_ref[:, 0:EPW] = src_ref[...]
    srcp_ref[:, EPW:EPWP] = fill
    dstp_ref[:, 0:EPW] = dst_ref[...]
    dstp_ref[:, EPW:EPWP] = fill


_t2 = pl.pallas_call(
    _t2_body,
    out_shape=(jax.ShapeDtypeStruct((NP, EMB), jnp.float32),
               jax.ShapeDtypeStruct((NP, 1), jnp.float32),
               jax.ShapeDtypeStruct((B, EMB), jnp.float32),
               jax.ShapeDtypeStruct((NW, EPWP), jnp.int32),
               jax.ShapeDtypeStruct((NW, EPWP), jnp.int32)))


def _tgru_body(seq_ref, wih_ref, whh_ref, bih_ref, bhh_ref, h_ref):
    wih = wih_ref[...]
    whh = whh_ref[...]
    bih = bih_ref[...]
    bhh = bhh_ref[...]

    def step(t, h):
        x_t = seq_ref[t]
        gi = x_t @ wih + bih
        gh = h @ whh + bhh
        r = jax.nn.sigmoid(gi[:, 0:EMB] + gh[:, 0:EMB])
        z = jax.nn.sigmoid(gi[:, EMB:2 * EMB] + gh[:, EMB:2 * EMB])
        n = jnp.tanh(gi[:, 2 * EMB:3 * EMB] + r * gh[:, 2 * EMB:3 * EMB])
        return (1.0 - z) * n + z * h

    h_ref[...] = lax.fori_loop(0, SEQ, step, jnp.zeros((B, EMB), jnp.float32))


_tgru = pl.pallas_call(
    _tgru_body, out_shape=jax.ShapeDtypeStruct((B, EMB), jnp.float32))


def _t5_body(seg_ref, cpart_ref, y_ref, dinvc_ref, deg_ref, g1b_ref,
             g2w_ref, g2b_ref, tabemb_ref, h_ref, fa_ref, fb_ref, fc_ref,
             fbias_ref, cw_ref, cb_ref, out_ref):
    seg = seg_ref[0] + seg_ref[1]
    h1 = jnp.maximum(dinvc_ref[...] * (seg + y_ref[...]) + g1b_ref[...], 0.0)
    ones_row = jnp.ones((1, NW), jnp.float32)
    dinv_row = lax.rsqrt(ones_row @ deg_ref[...] + 1.0)
    csum = ones_row @ cpart_ref[...] + dinv_row
    mask = lax.broadcasted_iota(jnp.int32, (1, NP), 1) < NNODES
    w2 = jnp.where(mask, dinv_row * csum, 0.0)
    gsum = w2 @ h1
    grow = (gsum @ g2w_ref[...]) * (1.0 / NNODES) + g2b_ref[...]
    fused = jnp.maximum(
        tabemb_ref[...] @ fa_ref[...] + grow @ fb_ref[...]
        + h_ref[...] @ fc_ref[...] + fbias_ref[...], 0.0)
    out_ref[...] = fused @ cw_ref[...] + cb_ref[...]


def kernel(tabular, x, edge_index, seq, W_tab, b_tab, g1_W, g1_b, g2_W, g2_b,
           emb_table, gru_W_ih, gru_W_hh, gru_b_ih, gru_b_hh, fusion_W,
           fusion_b, cls_W, cls_b):
    nc = cls_W.shape[1]
    src2 = edge_index[0].reshape(NW, EPW)
    dst2 = edge_index[1].reshape(NW, EPW)
    dstr = edge_index[1].reshape(NW, DGR, 16)
    seqt = jnp.transpose(seq).reshape(NW, SPW)

    deg_part, emb_rows = _sc_degemb(dstr, seqt, emb_table)
    y, dinv_col, tab_emb, srcp, dstp = _t2(
        x, g1_W, deg_part, tabular, W_tab, b_tab.reshape(1, EMB), src2, dst2)
    seg_part, c_part = _sc_edges(
        srcp, dstp.reshape(NW, NCH, CHUNK), y, dinv_col.reshape(NP))

    seq_t = emb_rows.reshape(SEQ, B, EMB)
    h = _tgru(seq_t, gru_W_ih.T, gru_W_hh.T,
              gru_b_ih.reshape(1, 3 * EMB), gru_b_hh.reshape(1, 3 * EMB))

    t5 = pl.pallas_call(
        _t5_body, out_shape=jax.ShapeDtypeStruct((B, nc), jnp.float32))
    logits = t5(seg_part, c_part, y, dinv_col, deg_part,
                g1_b.reshape(1, EMB), g2_W, g2_b.reshape(1, EMB),
                tab_emb, h,
                fusion_W[0:EMB], fusion_W[EMB:2 * EMB], fusion_W[2 * EMB:],
                fusion_b.reshape(1, EMB), cls_W, cls_b.reshape(1, nc))
    dummy = jnp.zeros((B, EMB), jnp.float32)
    return (logits, dummy)


# speedup vs baseline: 1.0077x; 1.0077x over previous
import functools

import jax
import jax.numpy as jnp
from jax import lax
from jax.experimental import pallas as pl
from jax.experimental.pallas import tpu as pltpu
from jax.experimental.pallas import tpu_sc as plsc

NNODES = 10000
NP = 10112
EMB = 32
NW = 32
EPW = 10000
CHUNK = 128
NCH = 80
EPWP = NCH * CHUNK
B = 1024
SEQ = 50
SPW = 1600
SCHUNK = 80
SNCH = SPW // SCHUNK
DGR = EPW // 16

_SC_MESH = plsc.VectorSubcoreMesh(
    core_axis_name="c", subcore_axis_name="s", num_cores=2, num_subcores=16)
_SC_PARAMS = pltpu.CompilerParams(
    needs_layout_passes=False, use_tc_tiling_on_sc=False)


@functools.partial(
    pl.kernel,
    out_type=(jax.ShapeDtypeStruct((NW, NP), jnp.float32),
              jax.ShapeDtypeStruct((B * SEQ, EMB), jnp.float32)),
    mesh=_SC_MESH,
    compiler_params=_SC_PARAMS,
    scratch_types=[pltpu.VMEM((DGR, 16), jnp.int32),
                   pltpu.VMEM((NP,), jnp.float32),
                   pltpu.VMEM((SPW,), jnp.int32),
                   pltpu.VMEM((SPW, EMB), jnp.float32),
                   pltpu.SemaphoreType.DMA])
def _sc_degemb(dstr_hbm, seqt_hbm, table_hbm, deg_hbm, emb_hbm,
               dst_v, deg_v, idx_v, rows_v, sem):
    c = lax.axis_index("c")
    s = lax.axis_index("s")
    wid = c * 16 + s

    pltpu.sync_copy(seqt_hbm.at[wid], idx_v)
    descs = [
        pltpu.async_copy(table_hbm.at[idx_v.at[pl.ds(k * SCHUNK, SCHUNK)]],
                         rows_v.at[pl.ds(k * SCHUNK, SCHUNK)], sem)
        for k in range(SNCH)
    ]

    def zero(i, carry):
        deg_v[pl.ds(i * 16, 16)] = jnp.zeros((16,), jnp.float32)
        return carry
    lax.fori_loop(0, NP // 16, zero, 0)

    pltpu.sync_copy(dstr_hbm.at[wid], dst_v)
    ones = jnp.ones((16,), jnp.float32)

    def body(g, carry):
        plsc.addupdate_scatter(deg_v, [dst_v[g]], ones)
        return carry
    lax.fori_loop(0, DGR, body, 0)
    pltpu.sync_copy(deg_v, deg_hbm.at[wid])

    for d in descs:
        d.wait()
    pltpu.sync_copy(rows_v, emb_hbm.at[pl.ds(wid * SPW, SPW)])


@functools.partial(
    pl.kernel,
    out_type=(jax.ShapeDtypeStruct((2, NP, EMB), jnp.float32),
              jax.ShapeDtypeStruct((NW, NP), jnp.float32)),
    mesh=_SC_MESH,
    compiler_params=_SC_PARAMS,
    scratch_types=[
        pltpu.VMEM((EPWP,), jnp.int32),
        pltpu.VMEM((NCH, CHUNK), jnp.int32),
        pltpu.VMEM((NP,), jnp.float32),
        pltpu.VMEM((NP,), jnp.float32),
        pltpu.VMEM((CHUNK, EMB), jnp.float32),
        pltpu.VMEM((CHUNK, EMB), jnp.float32),
        pltpu.VMEM((CHUNK, EMB), jnp.float32),
        pltpu.VMEM((CHUNK, EMB), jnp.float32),
        pltpu.VMEM((NP // 16, EMB), jnp.float32),
        pltpu.VMEM_SHARED((NP, EMB), jnp.float32),
        pltpu.SemaphoreType.DMA, pltpu.SemaphoreType.DMA,
        pltpu.SemaphoreType.DMA, pltpu.SemaphoreType.DMA,
        pltpu.SemaphoreType.DMA, pltpu.SemaphoreType.DMA,
        pltpu.SemaphoreType.DMA, pltpu.SemaphoreType.DMA,
    ])
def _sc_edges(srcf_hbm, dstt_hbm, y_hbm, dinv_hbm,
              seg_hbm, cpart_hbm,
              src_v, dstt_v, dinv_v, c_v, r0, r1, r2, r3, dr_v, seg_sh,
              g0, g1, g2, g3, s0, s1, s2, s3):
    c = lax.axis_index("c")
    s = lax.axis_index("s")
    wid = c * 16 + s
    npt = NP // 16
    bufs = (r0, r1, r2, r3)
    gsems = (g0, g1, g2, g3)
    ssems = (s0, s1, s2, s3)

    d_src = pltpu.async_copy(srcf_hbm.at[wid], src_v, g0)
    d_dst = pltpu.async_copy(dstt_hbm.at[wid], dstt_v, g1)
    d_dnv = pltpu.async_copy(dinv_hbm, dinv_v, g2)

    def zero_c(i, carry):
        c_v[pl.ds(i * 16, 16)] = jnp.zeros((16,), jnp.float32)
        return carry
    lax.fori_loop(0, NP // 16, zero_c, 0)

    def zero_dr(i, carry):
        dr_v[i // 2, pl.ds((i % 2) * 16, 16)] = jnp.zeros((16,), jnp.float32)
        return carry
    lax.fori_loop(0, npt * 2, zero_dr, 0)
    pltpu.sync_copy(dr_v, seg_sh.at[pl.ds(s * npt, npt)])

    d_src.wait()
    d_dst.wait()
    d_dnv.wait()
    plsc.subcore_barrier()

    for b in range(2):
        pltpu.async_copy(
            y_hbm.at[src_v.at[pl.ds(b * CHUNK, CHUNK)]], bufs[b], gsems[b])

    def quad(k, carry):
        for b in range(4):
            j = k * 4 + b
            bn = (b + 2) % 4
            pltpu.make_async_copy(
                y_hbm.at[src_v.at[pl.ds(j * CHUNK, CHUNK)]],
                bufs[b], gsems[b]).wait()
            pltpu.make_async_copy(
                bufs[b], seg_sh.at[dstt_v.at[j]], ssems[b]).start(add=True)
            for g in range(CHUNK // 16):
                base = j * CHUNK + g * 16
                di = plsc.load_gather(dinv_v, [dstt_v[j, pl.ds(g * 16, 16)]])
                plsc.addupdate_scatter(c_v, [src_v[pl.ds(base, 16)]], di)

            @pl.when(j >= 2)
            def _():
                pltpu.make_async_copy(
                    bufs[bn], seg_sh.at[dstt_v.at[j - 2]], ssems[bn]).wait()

            @pl.when(j + 2 < NCH)
            def _():
                pltpu.async_copy(
                    y_hbm.at[src_v.at[pl.ds((j + 2) * CHUNK, CHUNK)]],
                    bufs[bn], gsems[bn])
        return carry
    lax.fori_loop(0, NCH // 4, quad, 0)

    for b in range(2, 4):
        pltpu.make_async_copy(
            bufs[b], seg_sh.at[dstt_v.at[NCH - 4 + b]], ssems[b]).wait()

    plsc.subcore_barrier()
    pltpu.sync_copy(seg_sh.at[pl.ds(s * npt, npt)], dr_v)
    pltpu.sync_copy(dr_v, seg_hbm.at[c, pl.ds(s * npt, npt)])
    pltpu.sync_copy(c_v, cpart_hbm.at[wid])


def _t2_body(x_ref, w_ref, deg_ref, tab_ref, wt_ref, bt_ref,
             src_ref, dst_ref,
             y_ref, dinv_ref, tabemb_ref, srcp_ref, dstp_ref):
    ones = jnp.ones((NW, 1), jnp.float32)
    deg_col = lax.dot_general(
        deg_ref[...], ones, (((0,), (0,)), ((), ()))) + 1.0
    dinv_col = lax.rsqrt(deg_col)
    dinv_ref[...] = dinv_col
    xw = x_ref[...] @ w_ref[...]
    y_ref[0:NNODES, :] = xw * dinv_col[0:NNODES]
    y_ref[NNODES:NP, :] = jnp.zeros((NP - NNODES, EMB), jnp.float32)
    tabemb_ref[...] = jnp.maximum(tab_ref[...] @ wt_ref[...] + bt_ref[...], 0.0)
    fill = jnp.full((NW, EPWP - EPW), NNODES, jnp.int32)
    srcp_ref[:, 0:EPW] = src_ref[...]
    srcp_ref[:, EPW:EPWP] = fill
    dstp_ref[:, 0:EPW] = dst_ref[...]
    dstp_ref[:, EPW:EPWP] = fill


_t2 = pl.pallas_call(
    _t2_body,
    out_shape=(jax.ShapeDtypeStruct((NP, EMB), jnp.float32),
               jax.ShapeDtypeStruct((NP, 1), jnp.float32),
               jax.ShapeDtypeStruct((B, EMB), jnp.float32),
               jax.ShapeDtypeStruct((NW, EPWP), jnp.int32),
               jax.ShapeDtypeStruct((NW, EPWP), jnp.int32)))


def _tgru_body(seq_ref, wih_ref, whh_ref, bih_ref, bhh_ref, h_ref):
    wih = wih_ref[...]
    whh = whh_ref[...]
    bih = bih_ref[...]
    bhh = bhh_ref[...]
    dn = (((1,), (1,)), ((), ()))

    def step(t, h):
        x_t = seq_ref[:, t, :]
        gi = lax.dot_general(x_t, wih, dn) + bih
        gh = lax.dot_general(h, whh, dn) + bhh
        r = jax.nn.sigmoid(gi[:, 0:EMB] + gh[:, 0:EMB])
        z = jax.nn.sigmoid(gi[:, EMB:2 * EMB] + gh[:, EMB:2 * EMB])
        n = jnp.tanh(gi[:, 2 * EMB:3 * EMB] + r * gh[:, 2 * EMB:3 * EMB])
        return (1.0 - z) * n + z * h

    h_ref[...] = lax.fori_loop(0, SEQ, step, jnp.zeros((B, EMB), jnp.float32))


_tgru = pl.pallas_call(
    _tgru_body, out_shape=jax.ShapeDtypeStruct((B, EMB), jnp.float32))


def _t5_body(seg_ref, cpart_ref, y_ref, dinvc_ref, deg_ref, g1b_ref,
             g2w_ref, g2b_ref, tabemb_ref, h_ref, fw_ref,
             fbias_ref, cw_ref, cb_ref, out_ref, dummy_ref):
    seg = seg_ref[0] + seg_ref[1]
    h1 = jnp.maximum(dinvc_ref[...] * (seg + y_ref[...]) + g1b_ref[...], 0.0)
    ones_row = jnp.ones((1, NW), jnp.float32)
    dinv_row = lax.rsqrt(ones_row @ deg_ref[...] + 1.0)
    csum = ones_row @ cpart_ref[...] + dinv_row
    mask = lax.broadcasted_iota(jnp.int32, (1, NP), 1) < NNODES
    w2 = jnp.where(mask, dinv_row * csum, 0.0)
    gsum = w2 @ h1
    grow = (gsum @ g2w_ref[...]) * (1.0 / NNODES) + g2b_ref[...]
    fw = fw_ref[...]
    fused = jnp.maximum(
        tabemb_ref[...] @ fw[0:EMB] + grow @ fw[EMB:2 * EMB]
        + h_ref[...] @ fw[2 * EMB:3 * EMB] + fbias_ref[...], 0.0)
    out_ref[...] = fused @ cw_ref[...] + cb_ref[...]
    dummy_ref[...] = jnp.zeros((B, EMB), jnp.float32)


def kernel(tabular, x, edge_index, seq, W_tab, b_tab, g1_W, g1_b, g2_W, g2_b,
           emb_table, gru_W_ih, gru_W_hh, gru_b_ih, gru_b_hh, fusion_W,
           fusion_b, cls_W, cls_b):
    nc = cls_W.shape[1]
    src2 = edge_index[0].reshape(NW, EPW)
    dst2 = edge_index[1].reshape(NW, EPW)
    dstr = edge_index[1].reshape(NW, DGR, 16)
    seqt = seq.reshape(NW, SPW)

    deg_part, emb_rows = _sc_degemb(dstr, seqt, emb_table)
    y, dinv_col, tab_emb, srcp, dstp = _t2(
        x, g1_W, deg_part, tabular, W_tab, b_tab.reshape(1, EMB), src2, dst2)
    seg_part, c_part = _sc_edges(
        srcp, dstp.reshape(NW, NCH, CHUNK), y, dinv_col.reshape(NP))

    seq_bm = emb_rows.reshape(B, SEQ, EMB)
    h = _tgru(seq_bm, gru_W_ih, gru_W_hh,
              gru_b_ih.reshape(1, 3 * EMB), gru_b_hh.reshape(1, 3 * EMB))

    t5 = pl.pallas_call(
        _t5_body,
        out_shape=(jax.ShapeDtypeStruct((B, nc), jnp.float32),
                   jax.ShapeDtypeStruct((B, EMB), jnp.float32)))
    logits, dummy = t5(seg_part, c_part, y, dinv_col, deg_part,
                       g1_b.reshape(1, EMB), g2_W, g2_b.reshape(1, EMB),
                       tab_emb, h, fusion_W,
                       fusion_b.reshape(1, EMB), cls_W, cls_b.reshape(1, nc))
    return (logits, dummy)


# speedup vs baseline: 1.0220x; 1.0142x over previous
import functools

import jax
import jax.numpy as jnp
from jax import lax
from jax.experimental import pallas as pl
from jax.experimental.pallas import tpu as pltpu
from jax.experimental.pallas import tpu_sc as plsc

NNODES = 10000
NP = 10112
EMB = 32
NW = 32
EPW = 10000
CHUNK = 128
NCH = 80
EPWP = NCH * CHUNK
B = 1024
SEQ = 50
SPW = 1600
SCHUNK = 80
SNCH = SPW // SCHUNK
DGR = EPW // 16

_SC_MESH = plsc.VectorSubcoreMesh(
    core_axis_name="c", subcore_axis_name="s", num_cores=2, num_subcores=16)
_SC_PARAMS = pltpu.CompilerParams(
    needs_layout_passes=False, use_tc_tiling_on_sc=False)


@functools.partial(
    pl.kernel,
    out_type=jax.ShapeDtypeStruct((NW, NP), jnp.float32),
    mesh=_SC_MESH,
    compiler_params=_SC_PARAMS,
    scratch_types=[pltpu.VMEM((DGR, 16), jnp.int32),
                   pltpu.VMEM((NP,), jnp.float32),
                   pltpu.SemaphoreType.DMA])
def _sc_deg(dstr_hbm, deg_hbm, dst_v, deg_v, sem):
    c = lax.axis_index("c")
    s = lax.axis_index("s")
    wid = c * 16 + s

    d_dst = pltpu.async_copy(dstr_hbm.at[wid], dst_v, sem)

    def zero(i, carry):
        deg_v[pl.ds(i * 16, 16)] = jnp.zeros((16,), jnp.float32)
        return carry
    lax.fori_loop(0, NP // 16, zero, 0)
    d_dst.wait()
    ones = jnp.ones((16,), jnp.float32)

    def body(g, carry):
        plsc.addupdate_scatter(deg_v, [dst_v[g]], ones)
        return carry
    lax.fori_loop(0, DGR, body, 0)
    pltpu.sync_copy(deg_v, deg_hbm.at[wid])


@functools.partial(
    pl.kernel,
    out_type=jax.ShapeDtypeStruct((B * SEQ, EMB), jnp.float32),
    mesh=_SC_MESH,
    compiler_params=_SC_PARAMS,
    scratch_types=[pltpu.VMEM((SPW,), jnp.int32),
                   pltpu.VMEM((SPW, EMB), jnp.float32),
                   pltpu.SemaphoreType.DMA])
def _sc_emb(seqt_hbm, table_hbm, emb_hbm, idx_v, rows_v, sem):
    c = lax.axis_index("c")
    s = lax.axis_index("s")
    wid = c * 16 + s
    pltpu.sync_copy(seqt_hbm.at[wid], idx_v)
    descs = [
        pltpu.async_copy(table_hbm.at[idx_v.at[pl.ds(k * SCHUNK, SCHUNK)]],
                         rows_v.at[pl.ds(k * SCHUNK, SCHUNK)], sem)
        for k in range(SNCH)
    ]
    for d in descs:
        d.wait()
    pltpu.sync_copy(rows_v, emb_hbm.at[pl.ds(wid * SPW, SPW)])


@functools.partial(
    pl.kernel,
    out_type=(jax.ShapeDtypeStruct((2, NP, EMB), jnp.float32),
              jax.ShapeDtypeStruct((NW, NP), jnp.float32)),
    mesh=_SC_MESH,
    compiler_params=_SC_PARAMS,
    scratch_types=[
        pltpu.VMEM((EPWP,), jnp.int32),
        pltpu.VMEM((NCH, CHUNK), jnp.int32),
        pltpu.VMEM((NP,), jnp.float32),
        pltpu.VMEM((NP,), jnp.float32),
        pltpu.VMEM((CHUNK, EMB), jnp.float32),
        pltpu.VMEM((CHUNK, EMB), jnp.float32),
        pltpu.VMEM((CHUNK, EMB), jnp.float32),
        pltpu.VMEM((CHUNK, EMB), jnp.float32),
        pltpu.VMEM((NP // 16, EMB), jnp.float32),
        pltpu.VMEM_SHARED((NP, EMB), jnp.float32),
        pltpu.SemaphoreType.DMA, pltpu.SemaphoreType.DMA,
        pltpu.SemaphoreType.DMA, pltpu.SemaphoreType.DMA,
        pltpu.SemaphoreType.DMA, pltpu.SemaphoreType.DMA,
        pltpu.SemaphoreType.DMA, pltpu.SemaphoreType.DMA,
    ])
def _sc_edges(srcf_hbm, dstt_hbm, y_hbm, dinv_hbm,
              seg_hbm, cpart_hbm,
              src_v, dstt_v, dinv_v, c_v, r0, r1, r2, r3, dr_v, seg_sh,
              g0, g1, g2, g3, s0, s1, s2, s3):
    c = lax.axis_index("c")
    s = lax.axis_index("s")
    wid = c * 16 + s
    npt = NP // 16
    bufs = (r0, r1, r2, r3)
    gsems = (g0, g1, g2, g3)
    ssems = (s0, s1, s2, s3)

    d_src = pltpu.async_copy(srcf_hbm.at[wid], src_v, g0)
    d_dst = pltpu.async_copy(dstt_hbm.at[wid], dstt_v, g1)
    d_dnv = pltpu.async_copy(dinv_hbm, dinv_v, g2)

    def zero_c(i, carry):
        c_v[pl.ds(i * 16, 16)] = jnp.zeros((16,), jnp.float32)
        return carry
    lax.fori_loop(0, NP // 16, zero_c, 0)

    def zero_dr(i, carry):
        dr_v[i // 2, pl.ds((i % 2) * 16, 16)] = jnp.zeros((16,), jnp.float32)
        return carry
    lax.fori_loop(0, npt * 2, zero_dr, 0)
    pltpu.sync_copy(dr_v, seg_sh.at[pl.ds(s * npt, npt)])

    d_src.wait()
    d_dst.wait()
    d_dnv.wait()
    plsc.subcore_barrier()

    for b in range(2):
        pltpu.async_copy(
            y_hbm.at[src_v.at[pl.ds(b * CHUNK, CHUNK)]], bufs[b], gsems[b])

    def quad(k, carry):
        for b in range(4):
            j = k * 4 + b
            bn = (b + 2) % 4
            pltpu.make_async_copy(
                y_hbm.at[src_v.at[pl.ds(j * CHUNK, CHUNK)]],
                bufs[b], gsems[b]).wait()
            pltpu.make_async_copy(
                bufs[b], seg_sh.at[dstt_v.at[j]], ssems[b]).start(add=True)
            for g in range(CHUNK // 16):
                base = j * CHUNK + g * 16
                di = plsc.load_gather(dinv_v, [dstt_v[j, pl.ds(g * 16, 16)]])
                plsc.addupdate_scatter(c_v, [src_v[pl.ds(base, 16)]], di)

            @pl.when(j >= 2)
            def _():
                pltpu.make_async_copy(
                    bufs[bn], seg_sh.at[dstt_v.at[j - 2]], ssems[bn]).wait()

            @pl.when(j + 2 < NCH)
            def _():
                pltpu.async_copy(
                    y_hbm.at[src_v.at[pl.ds((j + 2) * CHUNK, CHUNK)]],
                    bufs[bn], gsems[bn])
        return carry
    lax.fori_loop(0, NCH // 4, quad, 0)

    for b in range(2, 4):
        pltpu.make_async_copy(
            bufs[b], seg_sh.at[dstt_v.at[NCH - 4 + b]], ssems[b]).wait()

    plsc.subcore_barrier()
    pltpu.sync_copy(seg_sh.at[pl.ds(s * npt, npt)], dr_v)
    pltpu.sync_copy(dr_v, seg_hbm.at[c, pl.ds(s * npt, npt)])
    pltpu.sync_copy(c_v, cpart_hbm.at[wid])


def _t2_body(x_ref, w_ref, deg_ref, tab_ref, wt_ref, bt_ref,
             src_ref, dst_ref,
             y_ref, dinv_ref, tabemb_ref, srcp_ref, dstp_ref):
    ones = jnp.ones((NW, 1), jnp.float32)
    deg_col = lax.dot_general(
        deg_ref[...], ones, (((0,), (0,)), ((), ()))) + 1.0
    dinv_col = lax.rsqrt(deg_col)
    dinv_ref[...] = dinv_col
    xw = x_ref[...] @ w_ref[...]
    y_ref[0:NNODES, :] = xw * dinv_col[0:NNODES]
    y_ref[NNODES:NP, :] = jnp.zeros((NP - NNODES, EMB), jnp.float32)
    tabemb_ref[...] = jnp.maximum(tab_ref[...] @ wt_ref[...] + bt_ref[...], 0.0)
    fill = jnp.full((NW, EPWP - EPW), NNODES, jnp.int32)
    srcp_ref[:, 0:EPW] = src_ref[...]
    srcp_ref[:, EPW:EPWP] = fill
    dstp_ref[:, 0:EPW] = dst_ref[...]
    dstp_ref[:, EPW:EPWP] = fill


_t2 = pl.pallas_call(
    _t2_body,
    out_shape=(jax.ShapeDtypeStruct((NP, EMB), jnp.float32),
               jax.ShapeDtypeStruct((NP, 1), jnp.float32),
               jax.ShapeDtypeStruct((B, EMB), jnp.float32),
               jax.ShapeDtypeStruct((NW, EPWP), jnp.int32),
               jax.ShapeDtypeStruct((NW, EPWP), jnp.int32)))


def _tgru_body(seq_ref, wih_ref, whh_ref, bih_ref, bhh_ref, h_ref):
    wih = wih_ref[...]
    whh = whh_ref[...]
    bih = bih_ref[...]
    bhh = bhh_ref[...]
    dn = (((1,), (1,)), ((), ()))

    def step(t, h):
        x_t = seq_ref[:, t, :]
        gi = lax.dot_general(x_t, wih, dn) + bih
        gh = lax.dot_general(h, whh, dn) + bhh
        r = jax.nn.sigmoid(gi[:, 0:EMB] + gh[:, 0:EMB])
        z = jax.nn.sigmoid(gi[:, EMB:2 * EMB] + gh[:, EMB:2 * EMB])
        n = jnp.tanh(gi[:, 2 * EMB:3 * EMB] + r * gh[:, 2 * EMB:3 * EMB])
        return (1.0 - z) * n + z * h

    h_ref[...] = lax.fori_loop(0, SEQ, step, jnp.zeros((B, EMB), jnp.float32))


_tgru = pl.pallas_call(
    _tgru_body, out_shape=jax.ShapeDtypeStruct((B, EMB), jnp.float32))


def _t5_body(seg_ref, cpart_ref, y_ref, dinvc_ref, deg_ref, g1b_ref,
             g2w_ref, g2b_ref, tabemb_ref, h_ref, fw_ref,
             fbias_ref, cw_ref, cb_ref, out_ref, dummy_ref):
    seg = seg_ref[0] + seg_ref[1]
    h1 = jnp.maximum(dinvc_ref[...] * (seg + y_ref[...]) + g1b_ref[...], 0.0)
    ones_row = jnp.ones((1, NW), jnp.float32)
    dinv_row = lax.rsqrt(ones_row @ deg_ref[...] + 1.0)
    csum = ones_row @ cpart_ref[...] + dinv_row
    mask = lax.broadcasted_iota(jnp.int32, (1, NP), 1) < NNODES
    w2 = jnp.where(mask, dinv_row * csum, 0.0)
    gsum = w2 @ h1
    grow = (gsum @ g2w_ref[...]) * (1.0 / NNODES) + g2b_ref[...]
    fw = fw_ref[...]
    fused = jnp.maximum(
        tabemb_ref[...] @ fw[0:EMB] + grow @ fw[EMB:2 * EMB]
        + h_ref[...] @ fw[2 * EMB:3 * EMB] + fbias_ref[...], 0.0)
    out_ref[...] = fused @ cw_ref[...] + cb_ref[...]
    dummy_ref[...] = jnp.zeros((B, EMB), jnp.float32)


def kernel(tabular, x, edge_index, seq, W_tab, b_tab, g1_W, g1_b, g2_W, g2_b,
           emb_table, gru_W_ih, gru_W_hh, gru_b_ih, gru_b_hh, fusion_W,
           fusion_b, cls_W, cls_b):
    nc = cls_W.shape[1]
    src2 = edge_index[0].reshape(NW, EPW)
    dst2 = edge_index[1].reshape(NW, EPW)
    dstr = edge_index[1].reshape(NW, DGR, 16)
    seqt = seq.reshape(NW, SPW)

    deg_part = _sc_deg(dstr)
    emb_rows = _sc_emb(seqt, emb_table)
    y, dinv_col, tab_emb, srcp, dstp = _t2(
        x, g1_W, deg_part, tabular, W_tab, b_tab.reshape(1, EMB), src2, dst2)
    seg_part, c_part = _sc_edges(
        srcp, dstp.reshape(NW, NCH, CHUNK), y, dinv_col.reshape(NP))

    seq_bm = emb_rows.reshape(B, SEQ, EMB)
    h = _tgru(seq_bm, gru_W_ih, gru_W_hh,
              gru_b_ih.reshape(1, 3 * EMB), gru_b_hh.reshape(1, 3 * EMB))

    t5 = pl.pallas_call(
        _t5_body,
        out_shape=(jax.ShapeDtypeStruct((B, nc), jnp.float32),
                   jax.ShapeDtypeStruct((B, EMB), jnp.float32)))
    logits, dummy = t5(seg_part, c_part, y, dinv_col, deg_part,
                       g1_b.reshape(1, EMB), g2_W, g2_b.reshape(1, EMB),
                       tab_emb, h, fusion_W,
                       fusion_b.reshape(1, EMB), cls_W, cls_b.reshape(1, nc))
    return (logits, dummy)
